# Initial kernel scaffold; baseline (speedup 1.0000x reference)
import jax
import jax.numpy as jnp
from jax.experimental import pallas as pl


def kernel(feature_indices, feature_values, embedding_weight):
    raise NotImplementedError("write your pallas kernel here")


# speedup vs baseline: 1.1918x; 1.1918x over previous
import functools

import jax
import jax.numpy as jnp
from jax import lax
from jax.experimental import pallas as pl
from jax.experimental.pallas import tpu as pltpu
from jax.experimental.pallas import tpu_sc as plsc

D = 16
F = 26
FPAD = 32
NW = 32
CHUNK = 64
ROWS = CHUNK * F


def _fm_sc(idx_hbm, vals_hbm, table_hbm, out_hbm, idx_v, rows_v, vals_v,
           out_v, sem):
    wid = lax.axis_index("s") * 2 + lax.axis_index("c")
    n_chunks = out_hbm.shape[0] // (NW * CHUNK)
    base = wid * (n_chunks * CHUNK)
    lane = lax.iota(jnp.int32, D)

    def chunk_body(c, carry):
        rowbase = base + c * CHUNK
        pltpu.sync_copy(idx_hbm.at[pl.ds(rowbase * F, ROWS)], idx_v)
        pltpu.async_copy(table_hbm.at[idx_v], rows_v, sem).wait()
        pltpu.sync_copy(vals_hbm.at[pl.ds(rowbase, CHUNK)], vals_v)

        def group_body(g, carry2):
            outvec = jnp.zeros((D,), jnp.float32)
            for rr in range(16):
                r = g * 16 + rr
                v0 = vals_v[r, 0:16]
                v1 = vals_v[r, 16:32]
                s = jnp.zeros((D,), jnp.float32)
                q = jnp.zeros((D,), jnp.float32)
                for f in range(F):
                    src = v0 if f < 16 else v1
                    fi = jnp.full((D,), f % 16, jnp.int32)
                    vf = src.at[fi].get(mode="promise_in_bounds")
                    w = vf * rows_v[r * F + f, :]
                    s = s + w
                    q = q + w * w
                t = s * s - q
                for sh in (8, 4, 2, 1):
                    t = t + t.at[lane ^ sh].get(mode="promise_in_bounds")
                outvec = jnp.where(lane == rr, 0.5 * t, outvec)
            out_v[pl.ds(g * 16, 16)] = outvec
            return carry2

        lax.fori_loop(0, CHUNK // 16, group_body, 0)
        pltpu.sync_copy(out_v, out_hbm.at[pl.ds(rowbase, CHUNK)])
        return carry

    lax.fori_loop(0, n_chunks, chunk_body, 0)


def kernel(feature_indices, feature_values, embedding_weight):
    b, f = feature_indices.shape
    idx_flat = feature_indices.reshape(-1).astype(jnp.int32)
    vals_pad = jnp.zeros((b, FPAD), jnp.float32).at[:, :f].set(feature_values)

    mesh = plsc.VectorSubcoreMesh(core_axis_name="c", subcore_axis_name="s")
    fm = functools.partial(
        pl.kernel,
        mesh=mesh,
        out_type=jax.ShapeDtypeStruct((b,), jnp.float32),
        scratch_types=[
            pltpu.VMEM((ROWS,), jnp.int32),
            pltpu.VMEM((ROWS, D), jnp.float32),
            pltpu.VMEM((CHUNK, FPAD), jnp.float32),
            pltpu.VMEM((CHUNK,), jnp.float32),
            pltpu.SemaphoreType.DMA,
        ],
        compiler_params=pltpu.CompilerParams(use_tc_tiling_on_sc=False),
    )(_fm_sc)
    out = fm(idx_flat, vals_pad, embedding_weight)
    return out.reshape(b, 1)
